# Initial kernel scaffold; baseline (speedup 1.0000x reference)
import functools

import jax
import jax.numpy as jnp
from jax import lax
from jax.experimental import pallas as pl
from jax.experimental.pallas import tpu as pltpu
from jax.experimental.pallas import tpu_sc as plsc

BATCH = 4096
HIST = 50
EMBED_DIM = 64
N = BATCH * HIST
NUM_WORKERS = 32
PER_WORKER = N // NUM_WORKERS
CHUNK = 128
NUM_CHUNKS = PER_WORKER // CHUNK

_mesh = plsc.VectorSubcoreMesh(core_axis_name="c", subcore_axis_name="s")


@functools.partial(
    pl.kernel,
    mesh=_mesh,
    out_type=jax.ShapeDtypeStruct((N, EMBED_DIM), jnp.float32),
    scratch_types=[
        pltpu.VMEM((NUM_CHUNKS, CHUNK), jnp.int32),
        pltpu.VMEM((CHUNK, EMBED_DIM), jnp.float32),
        pltpu.SemaphoreType.DMA,
    ],
)
def _gather_kernel(idx_hbm, table_hbm, out_hbm, idx_v, rows_v, sem):
    wid = lax.axis_index("s") * 2 + lax.axis_index("c")
    base = wid * PER_WORKER
    pltpu.sync_copy(idx_hbm.at[wid], idx_v)

    def body(j, carry):
        pltpu.async_copy(table_hbm.at[idx_v.at[j]], rows_v, sem).wait()
        pltpu.sync_copy(rows_v, out_hbm.at[pl.ds(base + j * CHUNK, CHUNK)])
        return carry

    lax.fori_loop(0, NUM_CHUNKS, body, 0, unroll=False)


def kernel(x, text_embedding_vectors):
    idx = x.reshape(NUM_WORKERS, NUM_CHUNKS, CHUNK)
    out = _gather_kernel(idx, text_embedding_vectors)
    return out.reshape(BATCH, HIST, EMBED_DIM)


# speedup vs baseline: 4.0857x; 4.0857x over previous
import functools

import jax
import jax.numpy as jnp
from jax import lax
from jax.experimental import pallas as pl
from jax.experimental.pallas import tpu as pltpu
from jax.experimental.pallas import tpu_sc as plsc

BATCH = 4096
HIST = 50
EMBED_DIM = 64
N = BATCH * HIST
NUM_WORKERS = 32
PER_WORKER = N // NUM_WORKERS
CHUNK = 128
NUM_CHUNKS = PER_WORKER // CHUNK

_mesh = plsc.VectorSubcoreMesh(core_axis_name="c", subcore_axis_name="s")


@functools.partial(
    pl.kernel,
    mesh=_mesh,
    out_type=jax.ShapeDtypeStruct((N, EMBED_DIM), jnp.float32),
    compiler_params=pltpu.CompilerParams(use_tc_tiling_on_sc=False),
    scratch_types=[
        pltpu.VMEM((NUM_CHUNKS, CHUNK), jnp.int32),
        pltpu.VMEM((CHUNK, EMBED_DIM), jnp.float32),
        pltpu.SemaphoreType.DMA,
    ],
)
def _gather_kernel(idx_hbm, table_hbm, out_hbm, idx_v, rows_v, sem):
    wid = lax.axis_index("s") * 2 + lax.axis_index("c")
    base = wid * PER_WORKER
    pltpu.sync_copy(idx_hbm.at[wid], idx_v)

    def body(j, carry):
        pltpu.async_copy(table_hbm.at[idx_v.at[j]], rows_v, sem).wait()
        pltpu.sync_copy(rows_v, out_hbm.at[pl.ds(base + j * CHUNK, CHUNK)])
        return carry

    lax.fori_loop(0, NUM_CHUNKS, body, 0, unroll=False)


def kernel(x, text_embedding_vectors):
    idx = x.reshape(NUM_WORKERS, NUM_CHUNKS, CHUNK)
    out = _gather_kernel(idx, text_embedding_vectors)
    return out.reshape(BATCH, HIST, EMBED_DIM)


# speedup vs baseline: 4.6483x; 1.1377x over previous
import functools

import jax
import jax.numpy as jnp
from jax import lax
from jax.experimental import pallas as pl
from jax.experimental.pallas import tpu as pltpu
from jax.experimental.pallas import tpu_sc as plsc

BATCH = 4096
HIST = 50
EMBED_DIM = 64
N = BATCH * HIST
NUM_WORKERS = 32
PER_WORKER = N // NUM_WORKERS
CHUNK = 400
NUM_CHUNKS = PER_WORKER // CHUNK
NBUF = 4
GAHEAD = 2

_mesh = plsc.VectorSubcoreMesh(core_axis_name="c", subcore_axis_name="s")


@functools.partial(
    pl.kernel,
    mesh=_mesh,
    out_type=jax.ShapeDtypeStruct((N, EMBED_DIM), jnp.float32),
    compiler_params=pltpu.CompilerParams(use_tc_tiling_on_sc=False),
    scratch_types=[
        pltpu.VMEM((NUM_CHUNKS, CHUNK), jnp.int32),
        pltpu.VMEM((NBUF, CHUNK, EMBED_DIM), jnp.float32),
        pltpu.SemaphoreType.DMA((NBUF,)),
        pltpu.SemaphoreType.DMA((NBUF,)),
    ],
)
def _gather_kernel(idx_hbm, table_hbm, out_hbm, idx_v, bufs, gsems, osems):
    wid = lax.axis_index("s") * 2 + lax.axis_index("c")
    base = wid * PER_WORKER
    pltpu.sync_copy(idx_hbm.at[wid], idx_v)

    gather = [None] * NBUF
    outcp = [None] * NBUF
    for j in range(GAHEAD):
        b = j % NBUF
        gather[b] = pltpu.async_copy(
            table_hbm.at[idx_v.at[j]], bufs.at[b], gsems.at[b])
    for j in range(NUM_CHUNKS):
        b = j % NBUF
        gather[b].wait()
        outcp[b] = pltpu.async_copy(
            bufs.at[b], out_hbm.at[pl.ds(base + j * CHUNK, CHUNK)], osems.at[b])
        nj = j + GAHEAD
        if nj < NUM_CHUNKS:
            nb = nj % NBUF
            if outcp[nb] is not None:
                outcp[nb].wait()
            gather[nb] = pltpu.async_copy(
                table_hbm.at[idx_v.at[nj]], bufs.at[nb], gsems.at[nb])
    for j in range(NUM_CHUNKS - NBUF, NUM_CHUNKS):
        outcp[j % NBUF].wait()


def kernel(x, text_embedding_vectors):
    idx = x.reshape(NUM_WORKERS, NUM_CHUNKS, CHUNK)
    out = _gather_kernel(idx, text_embedding_vectors)
    return out.reshape(BATCH, HIST, EMBED_DIM)


# speedup vs baseline: 6.8448x; 1.4725x over previous
import functools

import jax
import jax.numpy as jnp
from jax import lax
from jax.experimental import pallas as pl
from jax.experimental.pallas import tpu as pltpu
from jax.experimental.pallas import tpu_sc as plsc

BATCH = 4096
HIST = 50
EMBED_DIM = 64
PLANE_H = 56
PLANE_W = 128
NUM_WORKERS = 32
BROWS_PER_W = BATCH // NUM_WORKERS
BCHUNK = 8
NUM_CHUNKS = BROWS_PER_W // BCHUNK
NBUF = 2

_mesh = plsc.VectorSubcoreMesh(core_axis_name="c", subcore_axis_name="s")


@functools.partial(
    pl.kernel,
    mesh=_mesh,
    out_type=jax.ShapeDtypeStruct((BATCH, PLANE_H, PLANE_W), jnp.float32),
    compiler_params=pltpu.CompilerParams(use_tc_tiling_on_sc=False),
    scratch_types=[
        pltpu.VMEM((BROWS_PER_W, HIST), jnp.int32),
        pltpu.VMEM((NBUF, BCHUNK, HIST, EMBED_DIM), jnp.float32),
        pltpu.SemaphoreType.DMA((NBUF,)),
        pltpu.SemaphoreType.DMA((NBUF,)),
    ],
)
def _gather_kernel(idx_hbm, table_hbm, out_hbm, idx_v, stage, gsems, osems):
    wid = lax.axis_index("s") * 2 + lax.axis_index("c")
    pltpu.sync_copy(idx_hbm.at[pl.ds(wid * BROWS_PER_W, BROWS_PER_W), :], idx_v)
    brow0 = wid * BROWS_PER_W

    def pair(i, carry):
        gh = [[None] * BCHUNK for _ in range(NBUF)]
        for b in range(NBUF):
            j = NBUF * i + b
            for r in range(BCHUNK):
                gh[b][r] = pltpu.async_copy(
                    table_hbm.at[idx_v.at[j * BCHUNK + r]],
                    stage.at[b, r],
                    gsems.at[b])
        oh = [None] * NBUF
        for b in range(NBUF):
            j = NBUF * i + b
            for r in range(BCHUNK):
                gh[b][r].wait()
            oh[b] = pltpu.async_copy(
                stage.at[b],
                out_hbm.at[pl.ds(brow0 + j * BCHUNK, BCHUNK),
                           pl.ds(0, HIST), pl.ds(0, EMBED_DIM)],
                osems.at[b])
        for b in range(NBUF):
            oh[b].wait()
        return carry

    lax.fori_loop(0, NUM_CHUNKS // NBUF, pair, 0, unroll=False)


def kernel(x, text_embedding_vectors):
    y = _gather_kernel(x, text_embedding_vectors)
    return y[:, :HIST, :EMBED_DIM]


# speedup vs baseline: 6.8487x; 1.0006x over previous
import functools

import jax
import jax.numpy as jnp
from jax import lax
from jax.experimental import pallas as pl
from jax.experimental.pallas import tpu as pltpu
from jax.experimental.pallas import tpu_sc as plsc

BATCH = 4096
HIST = 50
EMBED_DIM = 64
PLANE_H = 56
PLANE_W = 128
NUM_WORKERS = 32
BROWS_PER_W = BATCH // NUM_WORKERS
PER_WORKER = BROWS_PER_W * HIST
BCHUNK = 8
NUM_CHUNKS = BROWS_PER_W // BCHUNK
NBUF = 2

_mesh = plsc.VectorSubcoreMesh(core_axis_name="c", subcore_axis_name="s")


@functools.partial(
    pl.kernel,
    mesh=_mesh,
    out_type=jax.ShapeDtypeStruct((BATCH, PLANE_H, PLANE_W), jnp.float32),
    compiler_params=pltpu.CompilerParams(use_tc_tiling_on_sc=False),
    scratch_types=[
        pltpu.VMEM((PER_WORKER,), jnp.int32),
        pltpu.VMEM((NBUF, BCHUNK * HIST, EMBED_DIM), jnp.float32),
        pltpu.SemaphoreType.DMA((NBUF,)),
        pltpu.SemaphoreType.DMA((NBUF,)),
    ],
)
def _gather_kernel(idx_hbm, table_hbm, out_hbm, idx_v, stage, gsems, osems):
    wid = lax.axis_index("s") * 2 + lax.axis_index("c")
    pltpu.sync_copy(idx_hbm.at[pl.ds(wid * PER_WORKER, PER_WORKER)], idx_v)
    brow0 = wid * BROWS_PER_W

    def pair(i, carry):
        gh = [None] * NBUF
        for b in range(NBUF):
            j = NBUF * i + b
            gh[b] = pltpu.async_copy(
                table_hbm.at[idx_v.at[pl.ds(j * BCHUNK * HIST, BCHUNK * HIST)]],
                stage.at[b],
                gsems.at[b])
        oh = [[None] * BCHUNK for _ in range(NBUF)]
        for b in range(NBUF):
            j = NBUF * i + b
            gh[b].wait()
            for r in range(BCHUNK):
                oh[b][r] = pltpu.async_copy(
                    stage.at[b, pl.ds(r * HIST, HIST)],
                    out_hbm.at[brow0 + j * BCHUNK + r,
                               pl.ds(0, HIST), pl.ds(0, EMBED_DIM)],
                    osems.at[b])
        for b in range(NBUF):
            for r in range(BCHUNK):
                oh[b][r].wait()
        return carry

    lax.fori_loop(0, NUM_CHUNKS // NBUF, pair, 0, unroll=False)


def kernel(x, text_embedding_vectors):
    y = _gather_kernel(x.reshape(-1), text_embedding_vectors)
    return y[:, :HIST, :EMBED_DIM]


# speedup vs baseline: 6.8495x; 1.0001x over previous
import functools

import jax
import jax.numpy as jnp
from jax import lax
from jax.experimental import pallas as pl
from jax.experimental.pallas import tpu as pltpu
from jax.experimental.pallas import tpu_sc as plsc

BATCH = 4096
HIST = 50
EMBED_DIM = 64
PLANE_H = 56
PLANE_W = 128
NUM_WORKERS = 32
BROWS_PER_W = BATCH // NUM_WORKERS
PER_WORKER = BROWS_PER_W * HIST
BCHUNK = 8
NUM_CHUNKS = BROWS_PER_W // BCHUNK
NBUF = 4

_mesh = plsc.VectorSubcoreMesh(core_axis_name="c", subcore_axis_name="s")


@functools.partial(
    pl.kernel,
    mesh=_mesh,
    out_type=jax.ShapeDtypeStruct((BATCH, PLANE_H, PLANE_W), jnp.float32),
    compiler_params=pltpu.CompilerParams(use_tc_tiling_on_sc=False),
    scratch_types=[
        pltpu.VMEM((PER_WORKER,), jnp.int32),
        pltpu.VMEM((NBUF, BCHUNK * HIST, EMBED_DIM), jnp.float32),
        pltpu.SemaphoreType.DMA((NBUF,)),
        pltpu.SemaphoreType.DMA((NBUF,)),
    ],
)
def _gather_kernel(idx_hbm, table_hbm, out_hbm, idx_v, stage, gsems, osems):
    wid = lax.axis_index("s") * 2 + lax.axis_index("c")
    pltpu.sync_copy(idx_hbm.at[pl.ds(wid * PER_WORKER, PER_WORKER)], idx_v)
    brow0 = wid * BROWS_PER_W

    def _wait_outs(b):
        for r in range(BCHUNK):
            pltpu.make_async_copy(
                stage.at[b, pl.ds(r * HIST, HIST)],
                out_hbm.at[brow0, pl.ds(0, HIST), pl.ds(0, EMBED_DIM)],
                osems.at[b]).wait()

    def group(g, carry):
        gh = [None] * NBUF
        for b in range(NBUF):
            j = NBUF * g + b

            @pl.when(g > 0)
            def _(b=b):
                _wait_outs(b)

            gh[b] = pltpu.async_copy(
                table_hbm.at[idx_v.at[pl.ds(j * BCHUNK * HIST, BCHUNK * HIST)]],
                stage.at[b],
                gsems.at[b])
        for b in range(NBUF):
            j = NBUF * g + b
            gh[b].wait()
            for r in range(BCHUNK):
                pltpu.async_copy(
                    stage.at[b, pl.ds(r * HIST, HIST)],
                    out_hbm.at[brow0 + j * BCHUNK + r,
                               pl.ds(0, HIST), pl.ds(0, EMBED_DIM)],
                    osems.at[b])
        return carry

    lax.fori_loop(0, NUM_CHUNKS // NBUF, group, 0, unroll=False)
    for b in range(NBUF):
        _wait_outs(b)


def kernel(x, text_embedding_vectors):
    y = _gather_kernel(x.reshape(-1), text_embedding_vectors)
    return y[:, :HIST, :EMBED_DIM]
